# baseline (speedup 1.0000x reference)
import functools

import jax
import jax.numpy as jnp
from jax import lax
from jax.experimental import pallas as pl
from jax.experimental.pallas import tpu as pltpu
from jax.experimental.pallas import tpu_sc as plsc

DIM = 128
CHUNK = 128

try:
    _info = plsc.get_sparse_core_info()
    NC, NS = _info.num_cores, _info.num_subcores
except Exception:
    NC, NS = 2, 16
NW = NC * NS


def _sc_gather_pack(table, ids3):
    nw, nchunk, chunk = ids3.shape
    n = nw * nchunk * chunk
    per_w = n // NW
    half = chunk // 2
    mesh = plsc.VectorSubcoreMesh(core_axis_name="c", subcore_axis_name="s")

    nbuf = 4
    assert nchunk >= nbuf

    @functools.partial(
        pl.kernel,
        out_type=jax.ShapeDtypeStruct((n // 2, DIM), jnp.int32),
        mesh=mesh,
        scratch_types=[
            pltpu.VMEM((nchunk, CHUNK), jnp.int32),
            [pltpu.VMEM((CHUNK, DIM), jnp.float32) for _ in range(nbuf)],
            [pltpu.VMEM((CHUNK // 2, DIM), jnp.int32) for _ in range(nbuf)],
            [pltpu.SemaphoreType.DMA for _ in range(nbuf)],
            [pltpu.SemaphoreType.DMA for _ in range(nbuf)],
        ],
    )
    def gather_kernel(table_hbm, ids_hbm, out_hbm, idx_v, bufs, bufis, sems, ssems):
        wid = lax.axis_index("s") * NC + lax.axis_index("c")
        base2 = wid * (per_w // 2)
        pltpu.sync_copy(ids_hbm.at[wid], idx_v)

        def start(j, k):
            pltpu.make_async_copy(
                table_hbm.at[idx_v.at[j]], bufs[k], sems[k]
            ).start()

        def store_copy(j, k):
            return pltpu.make_async_copy(
                bufis[k], out_hbm.at[pl.ds(base2 + j * half, half)], ssems[k]
            )

        def pack(k):
            buf, bufi = bufs[k], bufis[k]

            def pack_row(t, carry):
                for cs in range(DIM // 16):
                    a = buf[t, pl.ds(cs * 16, 16)]
                    c = buf[t + half, pl.ds(cs * 16, 16)]
                    za = lax.shift_right_logical(
                        lax.bitcast_convert_type(a, jnp.int32)
                        + jnp.int32(0x8000),
                        16,
                    )
                    zc = (
                        lax.bitcast_convert_type(c, jnp.int32)
                        + jnp.int32(0x8000)
                    ) & jnp.int32(-65536)
                    bufi[t, pl.ds(cs * 16, 16)] = za | zc
                return carry

            lax.fori_loop(0, half, pack_row, 0)

        for j in range(nbuf):
            start(j, j)
        for j in range(nchunk):
            k = j % nbuf
            pltpu.make_async_copy(
                table_hbm.at[idx_v.at[j]], bufs[k], sems[k]
            ).wait()
            if j >= nbuf:
                store_copy(j - nbuf, k).wait()
            pack(k)
            store_copy(j, k).start()
            if j + nbuf < nchunk:
                start(j + nbuf, k)
        for j in range(nchunk - nbuf, nchunk):
            store_copy(j, j % nbuf).wait()

    return gather_kernel(table, ids3)


L_BLK = 2
K = 5


def _mm_body(x_ref, w_ref, b_ref, o_ref, B):
    xi = x_ref[...]
    xlo = lax.bitcast_convert_type(
        lax.shift_left(xi, 16), jnp.float32
    ).astype(jnp.bfloat16)
    xhi = lax.bitcast_convert_type(
        xi & jnp.int32(-65536), jnp.float32
    ).astype(jnp.bfloat16)
    dn = (((1,), (1,)), ((), ()))
    wb = w_ref[...].astype(jnp.bfloat16)
    ylo = lax.dot_general(xlo, wb, dn, preferred_element_type=jnp.float32)
    yhi = lax.dot_general(xhi, wb, dn, preferred_element_type=jnp.float32)
    r2 = xi.shape[0]
    nch = r2 // 64
    y = jnp.concatenate(
        [ylo.reshape(nch, 1, 64, DIM), yhi.reshape(nch, 1, 64, DIM)],
        axis=1,
    ).reshape(2 * r2, DIM) + b_ref[...]
    o_ref[...] = y.reshape(L_BLK, B, DIM)


def _tc_project_chunk(gp, W, b, out_prev, B, L, l_off):
    lk = 2 * gp.shape[0] // B
    grid = (lk // L_BLK,)
    r2 = L_BLK * B // 2
    x_spec = pl.BlockSpec((r2, DIM), lambda i: (i, 0))
    w_spec = pl.BlockSpec((DIM, DIM), lambda i: (0, 0))
    b_spec = pl.BlockSpec((DIM,), lambda i: (0,))
    blk = l_off // L_BLK
    o_spec = pl.BlockSpec((L_BLK, B, DIM), lambda i: (blk + i, 0, 0))
    out_shape = jax.ShapeDtypeStruct((L, B, DIM), jnp.float32)

    def mm_first(x_ref, w_ref, b_ref, o_ref):
        _mm_body(x_ref, w_ref, b_ref, o_ref, B)

    if out_prev is None:
        return pl.pallas_call(
            mm_first,
            grid=grid,
            in_specs=[x_spec, w_spec, b_spec],
            out_specs=o_spec,
            out_shape=out_shape,
        )(gp, W, b)

    def mm_alias(prev_ref, x_ref, w_ref, b_ref, o_ref):
        _mm_body(x_ref, w_ref, b_ref, o_ref, B)

    return pl.pallas_call(
        mm_alias,
        grid=grid,
        in_specs=[
            pl.BlockSpec(memory_space=pltpu.MemorySpace.HBM),
            x_spec,
            w_spec,
            b_spec,
        ],
        out_specs=o_spec,
        out_shape=out_shape,
        input_output_aliases={0: 0},
    )(out_prev, gp, W, b)


def kernel(news_ids, news_categ, table, W, b):
    B, L = news_ids.shape
    n = B * L
    lk = L // K
    ids4 = jnp.transpose(news_ids).reshape(K, NW, n // (K * NW * CHUNK), CHUNK)
    ids4 = ids4.astype(jnp.int32)
    gps = [_sc_gather_pack(table, ids4[k]) for k in range(K)]
    out = None
    for k in range(K):
        out = _tc_project_chunk(gps[k], W, b, out, B, L, k * lk)
    return jnp.transpose(out, (1, 0, 2))


# speedup vs baseline: 1.0036x; 1.0036x over previous
import functools

import jax
import jax.numpy as jnp
from jax import lax
from jax.experimental import pallas as pl
from jax.experimental.pallas import tpu as pltpu
from jax.experimental.pallas import tpu_sc as plsc

DIM = 128
CHUNK = 128

try:
    _info = plsc.get_sparse_core_info()
    NC, NS = _info.num_cores, _info.num_subcores
except Exception:
    NC, NS = 2, 16
NW = NC * NS


def _sc_gather_pack(table, ids3):
    nw, nchunk, chunk = ids3.shape
    n = nw * nchunk * chunk
    per_w = n // NW
    half = chunk // 2
    mesh = plsc.VectorSubcoreMesh(core_axis_name="c", subcore_axis_name="s")

    nbuf = 4
    assert nchunk >= nbuf

    @functools.partial(
        pl.kernel,
        out_type=jax.ShapeDtypeStruct((n // 2, DIM), jnp.int32),
        mesh=mesh,
        scratch_types=[
            pltpu.VMEM((nchunk, CHUNK), jnp.int32),
            [pltpu.VMEM((CHUNK, DIM), jnp.float32) for _ in range(nbuf)],
            [pltpu.VMEM((CHUNK // 2, DIM), jnp.int32) for _ in range(nbuf)],
            [pltpu.SemaphoreType.DMA for _ in range(nbuf)],
            [pltpu.SemaphoreType.DMA for _ in range(nbuf)],
        ],
    )
    def gather_kernel(table_hbm, ids_hbm, out_hbm, idx_v, bufs, bufis, sems, ssems):
        wid = lax.axis_index("s") * NC + lax.axis_index("c")
        base2 = wid * (per_w // 2)
        pltpu.sync_copy(ids_hbm.at[wid], idx_v)

        def start(j, k):
            pltpu.make_async_copy(
                table_hbm.at[idx_v.at[j]], bufs[k], sems[k]
            ).start()

        def store_copy(j, k):
            return pltpu.make_async_copy(
                bufis[k], out_hbm.at[pl.ds(base2 + j * half, half)], ssems[k]
            )

        def pack(k):
            buf, bufi = bufs[k], bufis[k]

            def pack_row(t, carry):
                for cs in range(DIM // 16):
                    a = buf[t, pl.ds(cs * 16, 16)]
                    c = buf[t + half, pl.ds(cs * 16, 16)]
                    za = lax.shift_right_logical(
                        lax.bitcast_convert_type(a, jnp.int32), 16
                    )
                    zc = lax.bitcast_convert_type(c, jnp.int32) & jnp.int32(
                        -65536
                    )
                    bufi[t, pl.ds(cs * 16, 16)] = za | zc
                return carry

            lax.fori_loop(0, half, pack_row, 0)

        for j in range(nbuf):
            start(j, j)
        for j in range(nchunk):
            k = j % nbuf
            pltpu.make_async_copy(
                table_hbm.at[idx_v.at[j]], bufs[k], sems[k]
            ).wait()
            if j >= nbuf:
                store_copy(j - nbuf, k).wait()
            pack(k)
            store_copy(j, k).start()
            if j + nbuf < nchunk:
                start(j + nbuf, k)
        for j in range(nchunk - nbuf, nchunk):
            store_copy(j, j % nbuf).wait()

    return gather_kernel(table, ids3)


L_BLK = 2
K = 5


def _mm_body(x_ref, w_ref, b_ref, o_ref, B):
    xi = x_ref[...]
    xlo = lax.bitcast_convert_type(
        lax.shift_left(xi, 16), jnp.float32
    ).astype(jnp.bfloat16)
    xhi = lax.bitcast_convert_type(
        xi & jnp.int32(-65536), jnp.float32
    ).astype(jnp.bfloat16)
    dn = (((1,), (1,)), ((), ()))
    wb = w_ref[...].astype(jnp.bfloat16)
    ylo = lax.dot_general(xlo, wb, dn, preferred_element_type=jnp.float32)
    yhi = lax.dot_general(xhi, wb, dn, preferred_element_type=jnp.float32)
    r2 = xi.shape[0]
    nch = r2 // 64
    y = jnp.concatenate(
        [ylo.reshape(nch, 1, 64, DIM), yhi.reshape(nch, 1, 64, DIM)],
        axis=1,
    ).reshape(2 * r2, DIM) + b_ref[...]
    o_ref[...] = y.reshape(L_BLK, B, DIM)


def _tc_project_chunk(gp, W, b, out_prev, B, L, l_off):
    lk = 2 * gp.shape[0] // B
    grid = (lk // L_BLK,)
    r2 = L_BLK * B // 2
    x_spec = pl.BlockSpec((r2, DIM), lambda i: (i, 0))
    w_spec = pl.BlockSpec((DIM, DIM), lambda i: (0, 0))
    b_spec = pl.BlockSpec((DIM,), lambda i: (0,))
    blk = l_off // L_BLK
    o_spec = pl.BlockSpec((L_BLK, B, DIM), lambda i: (blk + i, 0, 0))
    out_shape = jax.ShapeDtypeStruct((L, B, DIM), jnp.float32)

    def mm_first(x_ref, w_ref, b_ref, o_ref):
        _mm_body(x_ref, w_ref, b_ref, o_ref, B)

    if out_prev is None:
        return pl.pallas_call(
            mm_first,
            grid=grid,
            in_specs=[x_spec, w_spec, b_spec],
            out_specs=o_spec,
            out_shape=out_shape,
        )(gp, W, b)

    def mm_alias(prev_ref, x_ref, w_ref, b_ref, o_ref):
        _mm_body(x_ref, w_ref, b_ref, o_ref, B)

    return pl.pallas_call(
        mm_alias,
        grid=grid,
        in_specs=[
            pl.BlockSpec(memory_space=pltpu.MemorySpace.HBM),
            x_spec,
            w_spec,
            b_spec,
        ],
        out_specs=o_spec,
        out_shape=out_shape,
        input_output_aliases={0: 0},
    )(out_prev, gp, W, b)


def kernel(news_ids, news_categ, table, W, b):
    B, L = news_ids.shape
    n = B * L
    lk = L // K
    ids4 = jnp.transpose(news_ids).reshape(K, NW, n // (K * NW * CHUNK), CHUNK)
    ids4 = ids4.astype(jnp.int32)
    gps = [_sc_gather_pack(table, ids4[k]) for k in range(K)]
    out = None
    for k in range(K):
        out = _tc_project_chunk(gps[k], W, b, out, B, L, k * lk)
    return jnp.transpose(out, (1, 0, 2))


# speedup vs baseline: 1.0151x; 1.0115x over previous
import functools

import jax
import jax.numpy as jnp
from jax import lax
from jax.experimental import pallas as pl
from jax.experimental.pallas import tpu as pltpu
from jax.experimental.pallas import tpu_sc as plsc

DIM = 128
CHUNK = 128

try:
    _info = plsc.get_sparse_core_info()
    NC, NS = _info.num_cores, _info.num_subcores
except Exception:
    NC, NS = 2, 16
NW = NC * NS


def _sc_gather_pack(table, ids3):
    nw, nchunk, chunk = ids3.shape
    n = nw * nchunk * chunk
    per_w = n // NW
    half = chunk // 2
    mesh = plsc.VectorSubcoreMesh(core_axis_name="c", subcore_axis_name="s")

    nbuf = 4
    assert nchunk >= nbuf

    @functools.partial(
        pl.kernel,
        out_type=jax.ShapeDtypeStruct((n // 2, DIM), jnp.int32),
        mesh=mesh,
        scratch_types=[
            pltpu.VMEM((nchunk, CHUNK), jnp.int32),
            [pltpu.VMEM((CHUNK, DIM), jnp.float32) for _ in range(nbuf)],
            [pltpu.VMEM((CHUNK // 2, DIM), jnp.int32) for _ in range(nbuf)],
            [pltpu.SemaphoreType.DMA for _ in range(nbuf)],
            [pltpu.SemaphoreType.DMA for _ in range(nbuf)],
        ],
    )
    def gather_kernel(table_hbm, ids_hbm, out_hbm, idx_v, bufs, bufis, sems, ssems):
        wid = lax.axis_index("s") * NC + lax.axis_index("c")
        base2 = wid * (per_w // 2)
        pltpu.sync_copy(ids_hbm.at[wid], idx_v)

        def start(j, k):
            pltpu.make_async_copy(
                table_hbm.at[idx_v.at[j]], bufs[k], sems[k]
            ).start()

        def store_copy(j, k):
            return pltpu.make_async_copy(
                bufis[k], out_hbm.at[pl.ds(base2 + j * half, half)], ssems[k]
            )

        def pack(k):
            buf, bufi = bufs[k], bufis[k]

            def pack_row(t, carry):
                for cs in range(DIM // 16):
                    a = buf[t, pl.ds(cs * 16, 16)]
                    c = buf[t + half, pl.ds(cs * 16, 16)]
                    za = lax.shift_right_logical(
                        lax.bitcast_convert_type(a, jnp.int32)
                        + jnp.int32(0x8000),
                        16,
                    )
                    zc = (
                        lax.bitcast_convert_type(c, jnp.int32)
                        + jnp.int32(0x8000)
                    ) & jnp.int32(-65536)
                    bufi[t, pl.ds(cs * 16, 16)] = za | zc
                return carry

            lax.fori_loop(0, half, pack_row, 0)

        for j in range(nbuf):
            start(j, j)
        for j in range(nchunk):
            k = j % nbuf
            pltpu.make_async_copy(
                table_hbm.at[idx_v.at[j]], bufs[k], sems[k]
            ).wait()
            if j >= nbuf:
                store_copy(j - nbuf, k).wait()
            pack(k)
            store_copy(j, k).start()
            if j + nbuf < nchunk:
                start(j + nbuf, k)
        for j in range(nchunk - nbuf, nchunk):
            store_copy(j, j % nbuf).wait()

    return gather_kernel(table, ids3)


L_BLK = 2
CHUNK_LS = (6, 12, 12, 14, 6)


def _mm_body(x_ref, w_ref, b_ref, o_ref, B):
    xi = x_ref[...]
    xlo = lax.bitcast_convert_type(
        lax.shift_left(xi, 16), jnp.float32
    ).astype(jnp.bfloat16)
    xhi = lax.bitcast_convert_type(
        xi & jnp.int32(-65536), jnp.float32
    ).astype(jnp.bfloat16)
    dn = (((1,), (1,)), ((), ()))
    wb = w_ref[...].astype(jnp.bfloat16)
    ylo = lax.dot_general(xlo, wb, dn, preferred_element_type=jnp.float32)
    yhi = lax.dot_general(xhi, wb, dn, preferred_element_type=jnp.float32)
    r2 = xi.shape[0]
    nch = r2 // 64
    y = jnp.concatenate(
        [ylo.reshape(nch, 1, 64, DIM), yhi.reshape(nch, 1, 64, DIM)],
        axis=1,
    ).reshape(2 * r2, DIM) + b_ref[...]
    o_ref[...] = y.reshape(L_BLK, B, DIM)


def _tc_project_chunk(gp, W, b, out_prev, B, L, l_off):
    lk = 2 * gp.shape[0] // B
    grid = (lk // L_BLK,)
    r2 = L_BLK * B // 2
    x_spec = pl.BlockSpec((r2, DIM), lambda i: (i, 0))
    w_spec = pl.BlockSpec((DIM, DIM), lambda i: (0, 0))
    b_spec = pl.BlockSpec((DIM,), lambda i: (0,))
    blk = l_off // L_BLK
    o_spec = pl.BlockSpec((L_BLK, B, DIM), lambda i: (blk + i, 0, 0))
    out_shape = jax.ShapeDtypeStruct((L, B, DIM), jnp.float32)

    def mm_first(x_ref, w_ref, b_ref, o_ref):
        _mm_body(x_ref, w_ref, b_ref, o_ref, B)

    if out_prev is None:
        return pl.pallas_call(
            mm_first,
            grid=grid,
            in_specs=[x_spec, w_spec, b_spec],
            out_specs=o_spec,
            out_shape=out_shape,
        )(gp, W, b)

    def mm_alias(prev_ref, x_ref, w_ref, b_ref, o_ref):
        _mm_body(x_ref, w_ref, b_ref, o_ref, B)

    return pl.pallas_call(
        mm_alias,
        grid=grid,
        in_specs=[
            pl.BlockSpec(memory_space=pltpu.MemorySpace.HBM),
            x_spec,
            w_spec,
            b_spec,
        ],
        out_specs=o_spec,
        out_shape=out_shape,
        input_output_aliases={0: 0},
    )(out_prev, gp, W, b)


def kernel(news_ids, news_categ, table, W, b):
    B, L = news_ids.shape
    n = B * L
    ids_t = jnp.transpose(news_ids).astype(jnp.int32)
    out = None
    l_off = 0
    gps = []
    for lk in CHUNK_LS:
        ids_k = lax.slice(ids_t, (l_off, 0), (l_off + lk, B))
        gps.append(_sc_gather_pack(table, ids_k.reshape(NW, lk, CHUNK)))
        l_off += lk
    out = None
    l_off = 0
    for lk, gp in zip(CHUNK_LS, gps):
        out = _tc_project_chunk(gp, W, b, out, B, L, l_off)
        l_off += lk
    return jnp.transpose(out, (1, 0, 2))


# speedup vs baseline: 1.0168x; 1.0017x over previous
import functools

import jax
import jax.numpy as jnp
from jax import lax
from jax.experimental import pallas as pl
from jax.experimental.pallas import tpu as pltpu
from jax.experimental.pallas import tpu_sc as plsc

DIM = 128
CHUNK = 128

try:
    _info = plsc.get_sparse_core_info()
    NC, NS = _info.num_cores, _info.num_subcores
except Exception:
    NC, NS = 2, 16
NW = NC * NS


def _sc_gather_pack(table, ids3):
    nw, nchunk, chunk = ids3.shape
    n = nw * nchunk * chunk
    per_w = n // NW
    half = chunk // 2
    mesh = plsc.VectorSubcoreMesh(core_axis_name="c", subcore_axis_name="s")

    nbuf = 4
    assert nchunk >= nbuf

    @functools.partial(
        pl.kernel,
        out_type=jax.ShapeDtypeStruct((n // 2, DIM), jnp.int32),
        mesh=mesh,
        scratch_types=[
            pltpu.VMEM((nchunk, CHUNK), jnp.int32),
            [pltpu.VMEM((CHUNK, DIM), jnp.float32) for _ in range(nbuf)],
            [pltpu.VMEM((CHUNK // 2, DIM), jnp.int32) for _ in range(nbuf)],
            [pltpu.SemaphoreType.DMA for _ in range(nbuf)],
            [pltpu.SemaphoreType.DMA for _ in range(nbuf)],
        ],
    )
    def gather_kernel(table_hbm, ids_hbm, out_hbm, idx_v, bufs, bufis, sems, ssems):
        wid = lax.axis_index("s") * NC + lax.axis_index("c")
        base2 = wid * (per_w // 2)
        pltpu.sync_copy(ids_hbm.at[wid], idx_v)

        def start(j, k):
            pltpu.make_async_copy(
                table_hbm.at[idx_v.at[j]], bufs[k], sems[k]
            ).start()

        def store_copy(j, k):
            return pltpu.make_async_copy(
                bufis[k], out_hbm.at[pl.ds(base2 + j * half, half)], ssems[k]
            )

        def pack(k):
            buf, bufi = bufs[k], bufis[k]

            def pack_row(t, carry):
                for cs in range(DIM // 16):
                    a = buf[t, pl.ds(cs * 16, 16)]
                    c = buf[t + half, pl.ds(cs * 16, 16)]
                    za = lax.shift_right_logical(
                        lax.bitcast_convert_type(a, jnp.int32)
                        + jnp.int32(0x8000),
                        16,
                    )
                    zc = (
                        lax.bitcast_convert_type(c, jnp.int32)
                        + jnp.int32(0x8000)
                    ) & jnp.int32(-65536)
                    bufi[t, pl.ds(cs * 16, 16)] = za | zc
                return carry

            lax.fori_loop(0, half, pack_row, 0)

        for j in range(nbuf):
            start(j, j)
        for j in range(nchunk):
            k = j % nbuf
            pltpu.make_async_copy(
                table_hbm.at[idx_v.at[j]], bufs[k], sems[k]
            ).wait()
            if j >= nbuf:
                store_copy(j - nbuf, k).wait()
            pack(k)
            store_copy(j, k).start()
            if j + nbuf < nchunk:
                start(j + nbuf, k)
        for j in range(nchunk - nbuf, nchunk):
            store_copy(j, j % nbuf).wait()

    return gather_kernel(table, ids3)


L_BLK = 2
CHUNK_LS = (6, 12, 12, 14, 6)


def _mm_body(x_ref, w_ref, b_ref, o_ref, B):
    xi = x_ref[...]
    xlo = lax.bitcast_convert_type(
        lax.shift_left(xi, 16), jnp.float32
    ).astype(jnp.bfloat16)
    xhi = lax.bitcast_convert_type(
        xi & jnp.int32(-65536), jnp.float32
    ).astype(jnp.bfloat16)
    dn = (((1,), (1,)), ((), ()))
    wb = w_ref[...].astype(jnp.bfloat16)
    ylo = lax.dot_general(xlo, wb, dn, preferred_element_type=jnp.float32)
    yhi = lax.dot_general(xhi, wb, dn, preferred_element_type=jnp.float32)
    r2 = xi.shape[0]
    nch = r2 // 64
    y = jnp.concatenate(
        [ylo.reshape(nch, 1, 64, DIM), yhi.reshape(nch, 1, 64, DIM)],
        axis=1,
    ).reshape(2 * r2, DIM) + b_ref[...]
    o_ref[...] = y.reshape(L_BLK, B, DIM)


def _tc_project_chunk(gp, W, b, out_prev, B, L, l_off):
    lk = 2 * gp.shape[0] // B
    grid = (lk // L_BLK,)
    r2 = L_BLK * B // 2
    x_spec = pl.BlockSpec((r2, DIM), lambda i: (i, 0))
    w_spec = pl.BlockSpec((DIM, DIM), lambda i: (0, 0))
    b_spec = pl.BlockSpec((DIM,), lambda i: (0,))
    blk = l_off // L_BLK
    o_spec = pl.BlockSpec((L_BLK, B, DIM), lambda i: (blk + i, 0, 0))
    out_shape = jax.ShapeDtypeStruct((L, B, DIM), jnp.float32)

    def mm_first(x_ref, w_ref, b_ref, o_ref):
        _mm_body(x_ref, w_ref, b_ref, o_ref, B)

    if out_prev is None:
        return pl.pallas_call(
            mm_first,
            grid=grid,
            in_specs=[x_spec, w_spec, b_spec],
            out_specs=o_spec,
            out_shape=out_shape,
        )(gp, W, b)

    def mm_alias(prev_ref, x_ref, w_ref, b_ref, o_ref):
        _mm_body(x_ref, w_ref, b_ref, o_ref, B)

    return pl.pallas_call(
        mm_alias,
        grid=grid,
        in_specs=[
            pl.BlockSpec(memory_space=pltpu.MemorySpace.HBM),
            x_spec,
            w_spec,
            b_spec,
        ],
        out_specs=o_spec,
        out_shape=out_shape,
        input_output_aliases={0: 0},
    )(out_prev, gp, W, b)


def kernel(news_ids, news_categ, table, W, b):
    B, L = news_ids.shape
    n = B * L
    ids_t = jnp.transpose(news_ids).astype(jnp.int32)
    l_off = 0
    gps = []
    for lk in CHUNK_LS:
        ids_k = lax.slice(ids_t, (l_off, 0), (l_off + lk, B))
        gps.append(_sc_gather_pack(table, ids_k.reshape(NW, lk, CHUNK)))
        l_off += lk
    out = None
    l_off = 0
    for lk, gp in zip(CHUNK_LS, gps):
        out = _tc_project_chunk(gp, W, b, out, B, L, l_off)
        l_off += lk
    return jnp.transpose(out, (1, 0, 2))
